# baseline (speedup 1.0000x reference)
import functools
import math

import numpy as np
import jax
import jax.numpy as jnp
from jax import lax
from jax.experimental import pallas as pl
from jax.experimental.pallas import tpu as pltpu
from jax.experimental.pallas import tpu_sc as plsc

L = 16
CHUNK = 8
BPAIR = 2


def _pe_table_np(T, d_model):
    pos = np.arange(T, dtype=np.float32)[:, None]
    div_term = np.exp(
        np.arange(0, d_model, 2, dtype=np.float32) * (-math.log(10000.0) / d_model)
    ).astype(np.float32)
    ang = (pos * div_term).astype(np.float32)
    pe = np.stack([np.sin(ang), np.cos(ang)], axis=-1).reshape(T, d_model)
    return pe.astype(np.float32)


@functools.partial(jax.jit, static_argnames=("B", "T", "D"))
def _sc_embed_add(x, emb_table, *, B, T, D):
    N = B * T
    info = plsc.get_sparse_core_info()
    NC, NS = info.num_cores, info.num_subcores
    NW = NC * NS
    t_per_w = T // NW
    tc_per_w = t_per_w // CHUNK
    half_t = t_per_w // 2
    chunks = [(tc, b0) for tc in range(tc_per_w) for b0 in range(0, B, BPAIR)]
    n_chunks = len(chunks)
    boundary = n_chunks // 2

    pe = jnp.asarray(_pe_table_np(T, D).reshape(-1))

    mesh = plsc.VectorSubcoreMesh(core_axis_name="c", subcore_axis_name="s")

    @functools.partial(
        pl.kernel,
        mesh=mesh,
        out_type=jax.ShapeDtypeStruct((N, D), jnp.float32),
        scratch_types=[
            pltpu.VMEM((B * t_per_w,), jnp.int32),
            pltpu.VMEM((half_t * D,), jnp.float32),
            pltpu.VMEM((BPAIR * CHUNK, D), jnp.float32),
            pltpu.VMEM((BPAIR * CHUNK, D), jnp.float32),
            pltpu.VMEM((BPAIR * CHUNK, D), jnp.float32),
            pltpu.VMEM((BPAIR * CHUNK, D), jnp.float32),
            pltpu.SemaphoreType.DMA,
            pltpu.SemaphoreType.DMA,
            pltpu.SemaphoreType.DMA,
            pltpu.SemaphoreType.DMA,
            pltpu.SemaphoreType.DMA,
        ],
    )
    def k(idx_hbm, table_hbm, pe_hbm, out_hbm, idx_v, pe_v, gb0, gb1, ob0, ob1,
          g0, g1, o0, o1, psem):
        wid = lax.axis_index("s") * NC + lax.axis_index("c")
        t0 = wid * t_per_w
        gbufs = (gb0, gb1)
        obufs = (ob0, ob1)
        gsems = (g0, g1)
        osems = (o0, o1)

        for b in range(B):
            pltpu.sync_copy(
                idx_hbm.at[b, pl.ds(t0, t_per_w)],
                idx_v.at[pl.ds(b * t_per_w, t_per_w)],
            )
        pe_cp = pltpu.async_copy(pe_hbm.at[pl.ds(t0 * D, half_t * D)], pe_v, psem)

        def issue_gather(ci, p):
            tc, b0 = chunks[ci]
            ds = []
            for h in range(BPAIR):
                ds.append(
                    pltpu.async_copy(
                        table_hbm.at[
                            idx_v.at[pl.ds((b0 + h) * t_per_w + tc * CHUNK, CHUNK)]
                        ],
                        gbufs[p].at[pl.ds(h * CHUNK, CHUNK)],
                        gsems[p],
                    )
                )
            return ds

        gathers = [None, None]
        gathers[0] = issue_gather(0, 0)
        gathers[1] = issue_gather(1, 1)
        pe_cp.wait()

        out_cps = [None, None]
        pe_restage = None
        for ci in range(n_chunks):
            p = ci % 2
            tc, b0 = chunks[ci]
            for d in gathers[p]:
                d.wait()
            if ci == boundary:
                pe_restage.wait()
            if out_cps[p] is not None:
                for d in out_cps[p]:
                    d.wait()
                out_cps[p] = None
            gbuf, obuf = gbufs[p], obufs[p]
            pe_row0 = (tc % (tc_per_w // 2)) * CHUNK

            def col_body(j, _, gbuf=gbuf, obuf=obuf, pe_row0=pe_row0):
                for r in range(CHUNK):
                    pv = pe_v[pl.ds((pe_row0 + r) * D + j * L, L)]
                    obuf[r, pl.ds(j * L, L)] = gbuf[r, pl.ds(j * L, L)] + pv
                    obuf[CHUNK + r, pl.ds(j * L, L)] = (
                        gbuf[CHUNK + r, pl.ds(j * L, L)] + pv
                    )
                return 0

            lax.fori_loop(0, D // L, col_body, 0, unroll=4)
            if ci + 2 < n_chunks:
                gathers[p] = issue_gather(ci + 2, p)
            if ci == boundary - 1:
                pe_restage = pltpu.async_copy(
                    pe_hbm.at[pl.ds((t0 + half_t) * D, half_t * D)], pe_v, psem
                )
            cps = []
            for h in range(BPAIR):
                row0 = (b0 + h) * T + t0 + tc * CHUNK
                cps.append(
                    pltpu.async_copy(
                        obuf.at[pl.ds(h * CHUNK, CHUNK)],
                        out_hbm.at[pl.ds(row0, CHUNK)],
                        osems[p],
                    )
                )
            out_cps[p] = cps
        for p in range(2):
            if out_cps[p] is not None:
                for d in out_cps[p]:
                    d.wait()

    return k(x, emb_table, pe)


def kernel(x, emb_table):
    B, T = x.shape
    V, D = emb_table.shape
    out = _sc_embed_add(x.astype(jnp.int32), emb_table, B=B, T=T, D=D)
    return out.reshape(B, T, D)


# speedup vs baseline: 1.1071x; 1.1071x over previous
import functools
import math

import numpy as np
import jax
import jax.numpy as jnp
from jax import lax
from jax.experimental import pallas as pl
from jax.experimental.pallas import tpu as pltpu
from jax.experimental.pallas import tpu_sc as plsc

L = 16
CHUNK = 8
BPAIR = 2


def _pe_table_np(T, d_model):
    pos = np.arange(T, dtype=np.float32)[:, None]
    div_term = np.exp(
        np.arange(0, d_model, 2, dtype=np.float32) * (-math.log(10000.0) / d_model)
    ).astype(np.float32)
    ang = (pos * div_term).astype(np.float32)
    pe = np.stack([np.sin(ang), np.cos(ang)], axis=-1).reshape(T, d_model)
    return pe.astype(np.float32)


@functools.partial(jax.jit, static_argnames=("B", "T", "D"))
def _sc_embed_add(x, emb_table, *, B, T, D):
    N = B * T
    info = plsc.get_sparse_core_info()
    NC, NS = info.num_cores, info.num_subcores
    NW = NC * NS
    t_per_w = T // NW
    tc_per_w = t_per_w // CHUNK
    half_t = t_per_w // 2
    chunks = [(tc, b0) for tc in range(tc_per_w) for b0 in range(0, B, BPAIR)]
    n_chunks = len(chunks)
    boundary = n_chunks // 2

    pe = jnp.asarray(_pe_table_np(T, D).reshape(-1))

    mesh = plsc.VectorSubcoreMesh(core_axis_name="c", subcore_axis_name="s")

    @functools.partial(
        pl.kernel,
        mesh=mesh,
        out_type=jax.ShapeDtypeStruct((N, D), jnp.float32),
        scratch_types=[
            pltpu.VMEM((B * t_per_w,), jnp.int32),
            pltpu.VMEM((half_t * D,), jnp.float32),
            pltpu.VMEM((BPAIR * CHUNK, D), jnp.float32),
            pltpu.VMEM((BPAIR * CHUNK, D), jnp.float32),
            pltpu.VMEM((BPAIR * CHUNK, D), jnp.float32),
            pltpu.VMEM((BPAIR * CHUNK, D), jnp.float32),
            pltpu.SemaphoreType.DMA,
            pltpu.SemaphoreType.DMA,
            pltpu.SemaphoreType.DMA,
            pltpu.SemaphoreType.DMA,
            pltpu.SemaphoreType.DMA,
        ],
    )
    def k(idx_hbm, table_hbm, pe_hbm, out_hbm, idx_v, pe_v, gb0, gb1, ob0, ob1,
          g0, g1, o0, o1, psem):
        wid = lax.axis_index("s") * NC + lax.axis_index("c")
        t0 = wid * t_per_w
        gbufs = (gb0, gb1)
        obufs = (ob0, ob1)
        gsems = (g0, g1)
        osems = (o0, o1)

        for b in range(B):
            pltpu.sync_copy(
                idx_hbm.at[b, pl.ds(t0, t_per_w)],
                idx_v.at[pl.ds(b * t_per_w, t_per_w)],
            )
        pe_cp = pltpu.async_copy(pe_hbm.at[pl.ds(t0 * D, half_t * D)], pe_v, psem)

        def issue_gather(ci, p):
            tc, b0 = chunks[ci]
            ds = []
            for h in range(BPAIR):
                ds.append(
                    pltpu.async_copy(
                        table_hbm.at[
                            idx_v.at[pl.ds((b0 + h) * t_per_w + tc * CHUNK, CHUNK)]
                        ],
                        gbufs[p].at[pl.ds(h * CHUNK, CHUNK)],
                        gsems[p],
                    )
                )
            return ds

        gathers = [None, None]
        gathers[0] = issue_gather(0, 0)
        gathers[1] = issue_gather(1, 1)
        pe_cp.wait()

        out_cps = [None, None]
        pe_restage = None
        for ci in range(n_chunks):
            p = ci % 2
            tc, b0 = chunks[ci]
            for d in gathers[p]:
                d.wait()
            if ci == boundary:
                pe_restage.wait()
            if out_cps[p] is not None:
                for d in out_cps[p]:
                    d.wait()
                out_cps[p] = None
            gbuf, obuf = gbufs[p], obufs[p]
            pe_row0 = (tc % (tc_per_w // 2)) * CHUNK

            def col_body(j, _, gbuf=gbuf, obuf=obuf, pe_row0=pe_row0):
                for r in range(CHUNK):
                    pv = pe_v[pl.ds((pe_row0 + r) * D + j * L, L)]
                    obuf[r, pl.ds(j * L, L)] = gbuf[r, pl.ds(j * L, L)] + pv
                    obuf[CHUNK + r, pl.ds(j * L, L)] = (
                        gbuf[CHUNK + r, pl.ds(j * L, L)] + pv
                    )
                return 0

            lax.fori_loop(0, D // L, col_body, 0, unroll=2)
            if ci + 2 < n_chunks:
                gathers[p] = issue_gather(ci + 2, p)
            if ci == boundary - 1:
                pe_restage = pltpu.async_copy(
                    pe_hbm.at[pl.ds((t0 + half_t) * D, half_t * D)], pe_v, psem
                )
            cps = []
            for h in range(BPAIR):
                row0 = (b0 + h) * T + t0 + tc * CHUNK
                cps.append(
                    pltpu.async_copy(
                        obuf.at[pl.ds(h * CHUNK, CHUNK)],
                        out_hbm.at[pl.ds(row0, CHUNK)],
                        osems[p],
                    )
                )
            out_cps[p] = cps
        for p in range(2):
            if out_cps[p] is not None:
                for d in out_cps[p]:
                    d.wait()

    return k(x, emb_table, pe)


def kernel(x, emb_table):
    B, T = x.shape
    V, D = emb_table.shape
    out = _sc_embed_add(x.astype(jnp.int32), emb_table, B=B, T=T, D=D)
    return out.reshape(B, T, D)


# speedup vs baseline: 1.2105x; 1.0933x over previous
import functools
import math

import numpy as np
import jax
import jax.numpy as jnp
from jax import lax
from jax.experimental import pallas as pl
from jax.experimental.pallas import tpu as pltpu
from jax.experimental.pallas import tpu_sc as plsc

L = 16
CHUNK = 8
BPAIR = 2


def _pe_table_np(T, d_model):
    pos = np.arange(T, dtype=np.float32)[:, None]
    div_term = np.exp(
        np.arange(0, d_model, 2, dtype=np.float32) * (-math.log(10000.0) / d_model)
    ).astype(np.float32)
    ang = (pos * div_term).astype(np.float32)
    pe = np.stack([np.sin(ang), np.cos(ang)], axis=-1).reshape(T, d_model)
    return pe.astype(np.float32)


@functools.partial(jax.jit, static_argnames=("B", "T", "D"))
def _sc_embed_add(x, emb_table, pe, *, B, T, D):
    N = B * T
    info = plsc.get_sparse_core_info()
    NC, NS = info.num_cores, info.num_subcores
    NW = NC * NS
    t_per_w = T // NW
    tc_per_w = t_per_w // CHUNK
    half_t = t_per_w // 2
    chunks = [(tc, b0) for tc in range(tc_per_w) for b0 in range(0, B, BPAIR)]
    n_chunks = len(chunks)
    boundary = n_chunks // 2

    mesh = plsc.VectorSubcoreMesh(core_axis_name="c", subcore_axis_name="s")

    @functools.partial(
        pl.kernel,
        mesh=mesh,
        out_type=jax.ShapeDtypeStruct((N, D), jnp.float32),
        scratch_types=[
            pltpu.VMEM((B * t_per_w,), jnp.int32),
            pltpu.VMEM((half_t, D), jnp.float32),
            pltpu.VMEM((BPAIR * CHUNK, D), jnp.float32),
            pltpu.VMEM((BPAIR * CHUNK, D), jnp.float32),
            pltpu.VMEM((BPAIR * CHUNK, D), jnp.float32),
            pltpu.VMEM((BPAIR * CHUNK, D), jnp.float32),
            pltpu.SemaphoreType.DMA,
            pltpu.SemaphoreType.DMA,
            pltpu.SemaphoreType.DMA,
            pltpu.SemaphoreType.DMA,
            pltpu.SemaphoreType.DMA,
        ],
    )
    def k(idx_hbm, table_hbm, pe_hbm, out_hbm, idx_v, pe_v, gb0, gb1, ob0, ob1,
          g0, g1, o0, o1, psem):
        wid = lax.axis_index("s") * NC + lax.axis_index("c")
        t0 = wid * t_per_w
        gbufs = (gb0, gb1)
        obufs = (ob0, ob1)
        gsems = (g0, g1)
        osems = (o0, o1)

        for b in range(B):
            pltpu.sync_copy(
                idx_hbm.at[b, pl.ds(t0, t_per_w)],
                idx_v.at[pl.ds(b * t_per_w, t_per_w)],
            )
        pe_cp = pltpu.async_copy(pe_hbm.at[pl.ds(t0, half_t)], pe_v, psem)

        def issue_gather(ci, p):
            tc, b0 = chunks[ci]
            ds = []
            for h in range(BPAIR):
                ds.append(
                    pltpu.async_copy(
                        table_hbm.at[
                            idx_v.at[pl.ds((b0 + h) * t_per_w + tc * CHUNK, CHUNK)]
                        ],
                        gbufs[p].at[pl.ds(h * CHUNK, CHUNK)],
                        gsems[p],
                    )
                )
            return ds

        gathers = [None, None]
        gathers[0] = issue_gather(0, 0)
        gathers[1] = issue_gather(1, 1)
        pe_cp.wait()

        out_cps = [None, None]
        pe_restage = None
        for ci in range(n_chunks):
            p = ci % 2
            tc, b0 = chunks[ci]
            for d in gathers[p]:
                d.wait()
            if ci == boundary:
                pe_restage.wait()
            if out_cps[p] is not None:
                for d in out_cps[p]:
                    d.wait()
                out_cps[p] = None
            gbuf, obuf = gbufs[p], obufs[p]
            pe_row0 = (tc % (tc_per_w // 2)) * CHUNK

            def col_body(j, _, gbuf=gbuf, obuf=obuf, pe_row0=pe_row0):
                for r in range(CHUNK):
                    pv = pe_v[pe_row0 + r, pl.ds(j * L, L)]
                    obuf[r, pl.ds(j * L, L)] = gbuf[r, pl.ds(j * L, L)] + pv
                    obuf[CHUNK + r, pl.ds(j * L, L)] = (
                        gbuf[CHUNK + r, pl.ds(j * L, L)] + pv
                    )
                return 0

            lax.fori_loop(0, D // L, col_body, 0, unroll=2)
            if ci + 2 < n_chunks:
                gathers[p] = issue_gather(ci + 2, p)
            if ci == boundary - 1:
                pe_restage = pltpu.async_copy(
                    pe_hbm.at[pl.ds(t0 + half_t, half_t)], pe_v, psem
                )
            cps = []
            for h in range(BPAIR):
                row0 = (b0 + h) * T + t0 + tc * CHUNK
                cps.append(
                    pltpu.async_copy(
                        obuf.at[pl.ds(h * CHUNK, CHUNK)],
                        out_hbm.at[pl.ds(row0, CHUNK)],
                        osems[p],
                    )
                )
            out_cps[p] = cps
        for p in range(2):
            if out_cps[p] is not None:
                for d in out_cps[p]:
                    d.wait()

    return k(x, emb_table, pe)


_PE_CACHE = {}


def _pe_device(T, D):
    key = (T, D)
    if key not in _PE_CACHE:
        _PE_CACHE[key] = jax.device_put(_pe_table_np(T, D))
    return _PE_CACHE[key]


def kernel(x, emb_table):
    B, T = x.shape
    V, D = emb_table.shape
    out = _sc_embed_add(x.astype(jnp.int32), emb_table, _pe_device(T, D), B=B, T=T, D=D)
    return out.reshape(B, T, D)


# speedup vs baseline: 1.2358x; 1.0209x over previous
import functools
import math

import numpy as np
import jax
import jax.numpy as jnp
from jax import lax
from jax.experimental import pallas as pl
from jax.experimental.pallas import tpu as pltpu
from jax.experimental.pallas import tpu_sc as plsc

L = 16
CHUNK = 8
BPAIR = 2
NGB = 3


def _pe_table_np(T, d_model):
    pos = np.arange(T, dtype=np.float32)[:, None]
    div_term = np.exp(
        np.arange(0, d_model, 2, dtype=np.float32) * (-math.log(10000.0) / d_model)
    ).astype(np.float32)
    ang = (pos * div_term).astype(np.float32)
    pe = np.stack([np.sin(ang), np.cos(ang)], axis=-1).reshape(T, d_model)
    return pe.astype(np.float32)


@functools.partial(jax.jit, static_argnames=("B", "T", "D"))
def _sc_embed_add(x, emb_table, pe, *, B, T, D):
    N = B * T
    info = plsc.get_sparse_core_info()
    NC, NS = info.num_cores, info.num_subcores
    NW = NC * NS
    t_per_w = T // NW
    tc_per_w = t_per_w // CHUNK
    half_t = t_per_w // 2
    chunks = [(tc, b0) for tc in range(tc_per_w) for b0 in range(0, B, BPAIR)]
    n_chunks = len(chunks)
    boundary = n_chunks // 2

    mesh = plsc.VectorSubcoreMesh(core_axis_name="c", subcore_axis_name="s")

    @functools.partial(
        pl.kernel,
        mesh=mesh,
        out_type=jax.ShapeDtypeStruct((N, D), jnp.float32),
        scratch_types=[
            pltpu.VMEM((B, t_per_w), jnp.int32),
            pltpu.VMEM((half_t, D), jnp.float32),
            pltpu.VMEM((BPAIR * CHUNK, D), jnp.float32),
            pltpu.VMEM((BPAIR * CHUNK, D), jnp.float32),
            pltpu.VMEM((BPAIR * CHUNK, D), jnp.float32),
            pltpu.VMEM((BPAIR * CHUNK, D), jnp.float32),
            pltpu.VMEM((BPAIR * CHUNK, D), jnp.float32),
            pltpu.SemaphoreType.DMA,
            pltpu.SemaphoreType.DMA,
            pltpu.SemaphoreType.DMA,
            pltpu.SemaphoreType.DMA,
            pltpu.SemaphoreType.DMA,
            pltpu.SemaphoreType.DMA,
        ],
    )
    def k(idx_hbm, table_hbm, pe_hbm, out_hbm, idx_v, pe_v,
          gb0, gb1, gb2, ob0, ob1, g0, g1, g2, o0, o1, psem):
        wid = lax.axis_index("s") * NC + lax.axis_index("c")
        t0 = wid * t_per_w
        gbufs = (gb0, gb1, gb2)
        obufs = (ob0, ob1)
        gsems = (g0, g1, g2)
        osems = (o0, o1)

        idx_sems = (g0, g1, g2, o0)
        idx_cps = [
            pltpu.async_copy(
                idx_hbm.at[b, pl.ds(t0, t_per_w)],
                idx_v.at[b],
                idx_sems[b % len(idx_sems)],
            )
            for b in range(B)
        ]
        pe_cp = pltpu.async_copy(pe_hbm.at[pl.ds(t0, half_t)], pe_v, psem)

        def issue_gather(ci):
            q = ci % NGB
            tc, b0 = chunks[ci]
            ds = []
            for h in range(BPAIR):
                ds.append(
                    pltpu.async_copy(
                        table_hbm.at[idx_v.at[b0 + h, pl.ds(tc * CHUNK, CHUNK)]],
                        gbufs[q].at[pl.ds(h * CHUNK, CHUNK)],
                        gsems[q],
                    )
                )
            return ds

        for cp in idx_cps:
            cp.wait()
        gathers = [None] * NGB
        gathers[0] = issue_gather(0)
        gathers[1] = issue_gather(1)
        pe_cp.wait()

        out_cps = [None, None]
        pe_restage = None
        for ci in range(n_chunks):
            q = ci % NGB
            p = ci % 2
            tc, b0 = chunks[ci]
            for d in gathers[q]:
                d.wait()
            if ci + 2 < n_chunks:
                gathers[(ci + 2) % NGB] = issue_gather(ci + 2)
            if ci == boundary:
                pe_restage.wait()
            if out_cps[p] is not None:
                for d in out_cps[p]:
                    d.wait()
                out_cps[p] = None
            gbuf, obuf = gbufs[q], obufs[p]
            pe_row0 = (tc % (tc_per_w // 2)) * CHUNK

            def col_body(j, _, gbuf=gbuf, obuf=obuf, pe_row0=pe_row0):
                for r in range(CHUNK):
                    pv = pe_v[pe_row0 + r, pl.ds(j * L, L)]
                    obuf[r, pl.ds(j * L, L)] = gbuf[r, pl.ds(j * L, L)] + pv
                    obuf[CHUNK + r, pl.ds(j * L, L)] = (
                        gbuf[CHUNK + r, pl.ds(j * L, L)] + pv
                    )
                return 0

            lax.fori_loop(0, D // L, col_body, 0, unroll=2)
            if ci == boundary - 1:
                pe_restage = pltpu.async_copy(
                    pe_hbm.at[pl.ds(t0 + half_t, half_t)], pe_v, psem
                )
            cps = []
            for h in range(BPAIR):
                row0 = (b0 + h) * T + t0 + tc * CHUNK
                cps.append(
                    pltpu.async_copy(
                        obuf.at[pl.ds(h * CHUNK, CHUNK)],
                        out_hbm.at[pl.ds(row0, CHUNK)],
                        osems[p],
                    )
                )
            out_cps[p] = cps
        for p in range(2):
            if out_cps[p] is not None:
                for d in out_cps[p]:
                    d.wait()

    return k(x, emb_table, pe)


_PE_CACHE = {}


def _pe_device(T, D):
    key = (T, D)
    if key not in _PE_CACHE:
        _PE_CACHE[key] = jax.device_put(_pe_table_np(T, D))
    return _PE_CACHE[key]


def kernel(x, emb_table):
    B, T = x.shape
    V, D = emb_table.shape
    out = _sc_embed_add(x.astype(jnp.int32), emb_table, _pe_device(T, D), B=B, T=T, D=D)
    return out.reshape(B, T, D)


# speedup vs baseline: 1.6796x; 1.3591x over previous
import functools
import math

import numpy as np
import jax
import jax.numpy as jnp
from jax import lax
from jax.experimental import pallas as pl
from jax.experimental.pallas import tpu as pltpu
from jax.experimental.pallas import tpu_sc as plsc

L = 16
CHUNK = 8
BPAIR = 2
NGB = 3


def _pe_table_np(T, d_model):
    pos = np.arange(T, dtype=np.float32)[:, None]
    div_term = np.exp(
        np.arange(0, d_model, 2, dtype=np.float32) * (-math.log(10000.0) / d_model)
    ).astype(np.float32)
    ang = (pos * div_term).astype(np.float32)
    pe = np.stack([np.sin(ang), np.cos(ang)], axis=-1).reshape(T, d_model)
    return pe.astype(np.float32)


@functools.partial(jax.jit, static_argnames=("B", "T", "D"))
def _sc_embed_add(x, emb_table, pe, *, B, T, D):
    N = B * T
    info = plsc.get_sparse_core_info()
    NC, NS = info.num_cores, info.num_subcores
    NW = NC * NS
    t_per_w = T // NW
    tc_per_w = t_per_w // CHUNK
    half_t = t_per_w // 2
    chunks = [(tc, b0) for tc in range(tc_per_w) for b0 in range(0, B, BPAIR)]
    n_chunks = len(chunks)
    boundary = n_chunks // 2

    mesh = plsc.VectorSubcoreMesh(core_axis_name="c", subcore_axis_name="s")

    @functools.partial(
        pl.kernel,
        mesh=mesh,
        out_type=jax.ShapeDtypeStruct((N, D), jnp.float32),
        scratch_types=[
            pltpu.VMEM((B, t_per_w), jnp.int32),
            pltpu.VMEM((half_t, D), jnp.float32),
            pltpu.VMEM((BPAIR * CHUNK, D), jnp.float32),
            pltpu.VMEM((BPAIR * CHUNK, D), jnp.float32),
            pltpu.VMEM((BPAIR * CHUNK, D), jnp.float32),
            pltpu.VMEM((BPAIR * CHUNK, D), jnp.float32),
            pltpu.VMEM((BPAIR * CHUNK, D), jnp.float32),
            pltpu.SemaphoreType.DMA,
            pltpu.SemaphoreType.DMA,
            pltpu.SemaphoreType.DMA,
            pltpu.SemaphoreType.DMA,
            pltpu.SemaphoreType.DMA,
            pltpu.SemaphoreType.DMA,
        ],
    )
    def k(idx_hbm, table_hbm, pe_hbm, out_hbm, idx_v, pe_v,
          gb0, gb1, gb2, ob0, ob1, g0, g1, g2, o0, o1, psem):
        wid = lax.axis_index("s") * NC + lax.axis_index("c")
        t0 = wid * t_per_w
        gbufs = (gb0, gb1, gb2)
        obufs = (ob0, ob1)
        gsems = (g0, g1, g2)
        osems = (o0, o1)

        idx_sems = (g0, g1, g2, o0)
        idx_cps = [
            pltpu.async_copy(
                idx_hbm.at[b, pl.ds(t0, t_per_w)],
                idx_v.at[b],
                idx_sems[b % len(idx_sems)],
            )
            for b in range(B)
        ]
        pe_cp = pltpu.async_copy(pe_hbm.at[pl.ds(t0, half_t)], pe_v, psem)

        def issue_gather(ci):
            q = ci % NGB
            tc, b0 = chunks[ci]
            ds = []
            for h in range(BPAIR):
                ds.append(
                    pltpu.async_copy(
                        table_hbm.at[idx_v.at[b0 + h, pl.ds(tc * CHUNK, CHUNK)]],
                        gbufs[q].at[pl.ds(h * CHUNK, CHUNK)],
                        gsems[q],
                    )
                )
            return ds

        for cp in idx_cps:
            cp.wait()
        gathers = [None] * NGB
        gathers[0] = issue_gather(0)
        gathers[1] = issue_gather(1)
        pe_cp.wait()

        out_cps = [None, None]
        pe_restage = None
        for ci in range(n_chunks):
            q = ci % NGB
            p = ci % 2
            tc, b0 = chunks[ci]
            for d in gathers[q]:
                d.wait()
            if ci + 2 < n_chunks:
                gathers[(ci + 2) % NGB] = issue_gather(ci + 2)
            if ci == boundary:
                pe_restage.wait()
            if out_cps[p] is not None:
                for d in out_cps[p]:
                    d.wait()
                out_cps[p] = None
            gbuf, obuf = gbufs[q], obufs[p]
            pe_row0 = (tc % (tc_per_w // 2)) * CHUNK

            @plsc.parallel_loop(0, D // L, unroll=2)
            def col_body(j, gbuf=gbuf, obuf=obuf, pe_row0=pe_row0):
                for r in range(CHUNK):
                    pv = pe_v[pe_row0 + r, pl.ds(j * L, L)]
                    obuf[r, pl.ds(j * L, L)] = gbuf[r, pl.ds(j * L, L)] + pv
                    obuf[CHUNK + r, pl.ds(j * L, L)] = (
                        gbuf[CHUNK + r, pl.ds(j * L, L)] + pv
                    )
            if ci == boundary - 1:
                pe_restage = pltpu.async_copy(
                    pe_hbm.at[pl.ds(t0 + half_t, half_t)], pe_v, psem
                )
            cps = []
            for h in range(BPAIR):
                row0 = (b0 + h) * T + t0 + tc * CHUNK
                cps.append(
                    pltpu.async_copy(
                        obuf.at[pl.ds(h * CHUNK, CHUNK)],
                        out_hbm.at[pl.ds(row0, CHUNK)],
                        osems[p],
                    )
                )
            out_cps[p] = cps
        for p in range(2):
            if out_cps[p] is not None:
                for d in out_cps[p]:
                    d.wait()

    return k(x, emb_table, pe)


_PE_CACHE = {}


def _pe_device(T, D):
    key = (T, D)
    if key not in _PE_CACHE:
        _PE_CACHE[key] = jax.device_put(_pe_table_np(T, D))
    return _PE_CACHE[key]


def kernel(x, emb_table):
    B, T = x.shape
    V, D = emb_table.shape
    out = _sc_embed_add(x.astype(jnp.int32), emb_table, _pe_device(T, D), B=B, T=T, D=D)
    return out.reshape(B, T, D)
